# Initial kernel scaffold; baseline (speedup 1.0000x reference)
import jax
import jax.numpy as jnp
from jax.experimental import pallas as pl


def kernel(y_pred, y_true, pos_idx, unlabeled_idx, edge_index):
    raise NotImplementedError("write your pallas kernel here")


# speedup vs baseline: 161.5471x; 161.5471x over previous
import functools

import jax
import jax.numpy as jnp
from jax import lax
from jax.experimental import pallas as pl
from jax.experimental.pallas import tpu as pltpu
from jax.experimental.pallas import tpu_sc as plsc

N_NODES = 100000
N_EDGES = 3200000
N_POS = 5000
N_UNL = 95000
PI_HAT = 0.6
PI_CHECK = 0.4

L = 16
NT = 16
NPAD = 102400
NODE_PER_TILE = NPAD // NT
E_PER_TILE = N_EDGES // NT
ECHUNK = 8000
NECHUNK = E_PER_TILE // ECHUNK
PPAD = 5008
UPAD = 95232
U_PER_TILE = UPAD // NT

_mesh = plsc.VectorSubcoreMesh(
    core_axis_name="c", subcore_axis_name="s", num_cores=1)


@functools.partial(
    pl.kernel,
    out_type=jax.ShapeDtypeStruct((L,), jnp.float32),
    mesh=_mesh,
    compiler_params=pltpu.CompilerParams(needs_layout_passes=False),
    scratch_types=[
        pltpu.VMEM((ECHUNK,), jnp.int32),
        pltpu.VMEM((ECHUNK,), jnp.int32),
        pltpu.VMEM((ECHUNK,), jnp.float32),
        pltpu.VMEM((PPAD,), jnp.int32),
        pltpu.VMEM((PPAD,), jnp.float32),
        pltpu.VMEM((NODE_PER_TILE,), jnp.float32),
        pltpu.VMEM((NODE_PER_TILE,), jnp.float32),
        pltpu.VMEM((U_PER_TILE,), jnp.int32),
        pltpu.VMEM((U_PER_TILE,), jnp.float32),
        pltpu.VMEM((U_PER_TILE,), jnp.float32),
        pltpu.VMEM((U_PER_TILE,), jnp.float32),
        pltpu.VMEM((4 * L,), jnp.float32),
        pltpu.VMEM((4 * L,), jnp.int32),
        pltpu.VMEM((8 * L,), jnp.float32),
        pltpu.VMEM((L,), jnp.float32),
        pltpu.VMEM_SHARED((NPAD,), jnp.float32),
        pltpu.VMEM_SHARED((NPAD,), jnp.float32),
        pltpu.VMEM_SHARED((NPAD,), jnp.float32),
        pltpu.VMEM_SHARED((8 * L,), jnp.float32),
    ],
)
def _sc_loss(ypred, posi, unli, edges, zeros, out,
             srcb, dstb, valb, pidx, pval, mb1, mb2,
             uidx, yub, f1b, f2b, accb, aidx, redb, outb,
             P, S1, S2, SACC):
    w = lax.axis_index("s")
    woff = w * NODE_PER_TILE
    z = jnp.zeros((L,), jnp.float32)

    nslc = pl.ds(woff, NODE_PER_TILE)
    pltpu.sync_copy(zeros.at[nslc], P.at[nslc])
    pltpu.sync_copy(zeros.at[nslc], S1.at[nslc])
    pltpu.sync_copy(zeros.at[nslc], S2.at[nslc])

    @pl.when(w == 0)
    def _():
        pltpu.sync_copy(zeros.at[pl.ds(0, 8 * L)], SACC)

    plsc.subcore_barrier()

    @pl.when(w == 0)
    def _():
        pltpu.sync_copy(posi, pidx)

        def fill(j, c):
            off = pl.multiple_of(j * L, L)
            pos = j * L + lax.iota(jnp.int32, L)
            pval[pl.ds(off, L)] = jnp.where(pos < N_POS, 1.0, 0.0)
            return c

        lax.fori_loop(0, PPAD // L, fill, 0)
        pltpu.sync_copy(pval, P.at[pidx], add=True)

    plsc.subcore_barrier()

    def edge_round(src_arr, dst_arr):
        ebase = w * E_PER_TILE

        def chunk(i, c):
            off = pl.multiple_of(ebase + i * ECHUNK, 8)
            offd = pl.multiple_of(N_EDGES + ebase + i * ECHUNK, 8)
            pltpu.sync_copy(edges.at[pl.ds(off, ECHUNK)], srcb)
            pltpu.sync_copy(edges.at[pl.ds(offd, ECHUNK)], dstb)
            pltpu.sync_copy(src_arr.at[srcb], valb)
            pltpu.sync_copy(valb, dst_arr.at[dstb], add=True)
            return c

        lax.fori_loop(0, NECHUNK, chunk, 0)

    edge_round(P, S1)
    plsc.subcore_barrier()

    pltpu.sync_copy(P.at[nslc], mb1)
    pltpu.sync_copy(S1.at[nslc], mb2)

    def addb(j, c):
        off = pl.multiple_of(j * L, L)
        mb1[pl.ds(off, L)] = mb1[pl.ds(off, L)] + mb2[pl.ds(off, L)]
        return c

    lax.fori_loop(0, NODE_PER_TILE // L, addb, 0)
    pltpu.sync_copy(mb1, S1.at[nslc])
    plsc.subcore_barrier()

    edge_round(S1, S2)
    plsc.subcore_barrier()

    ub = w * U_PER_TILE
    uslc = pl.ds(ub, U_PER_TILE)
    pltpu.sync_copy(unli.at[uslc], uidx)
    pltpu.sync_copy(ypred.at[uidx], yub)
    pltpu.sync_copy(S1.at[uidx], f1b)
    pltpu.sync_copy(S2.at[uidx], f2b)

    def lbody(j, carry):
        nc, nl, fl = carry
        off = pl.multiple_of(j * L, L)
        gpos = ub + j * L + lax.iota(jnp.int32, L)
        valid = gpos < N_UNL
        y = yub[pl.ds(off, L)]
        near = ((f1b[pl.ds(off, L)] + f2b[pl.ds(off, L)]) > 0.0) & valid
        far = valid & jnp.logical_not(near)
        nearf = jnp.where(near, 1.0, 0.0)
        farf = jnp.where(far, 1.0, 0.0)
        return (nc + nearf,
                nl + nearf * jnp.abs(y - PI_HAT),
                fl + farf * jnp.abs(y - PI_CHECK))

    nc, nl, fl = lax.fori_loop(0, U_PER_TILE // L, lbody, (z, z, z))
    accb[pl.ds(0, L)] = nc
    accb[pl.ds(L, L)] = nl
    accb[pl.ds(2 * L, L)] = fl
    accb[pl.ds(3 * L, L)] = z

    @pl.when(w == 0)
    def _():
        pltpu.sync_copy(ypred.at[pidx], pval)

        def pbody(j, carry):
            off = pl.multiple_of(j * L, L)
            pos = j * L + lax.iota(jnp.int32, L)
            vf = jnp.where(pos < N_POS, 1.0, 0.0)
            return carry + vf * jnp.abs(pval[pl.ds(off, L)] - 1.0)

        accb[pl.ds(3 * L, L)] = lax.fori_loop(0, PPAD // L, pbody, z)

    def ibody(j, c):
        off = pl.multiple_of(j * L, L)
        aidx[pl.ds(off, L)] = j * L + lax.iota(jnp.int32, L)
        return c

    lax.fori_loop(0, 4, ibody, 0)
    pltpu.sync_copy(accb, SACC.at[aidx], add=True)
    plsc.subcore_barrier()

    @pl.when(w == 0)
    def _():
        pltpu.sync_copy(SACC, redb)
        ncs = z + jnp.sum(redb[pl.ds(0, L)])
        nls = z + jnp.sum(redb[pl.ds(L, L)])
        fls = z + jnp.sum(redb[pl.ds(2 * L, L)])
        lls = z + jnp.sum(redb[pl.ds(3 * L, L)])
        loss = (2.0 * (PI_HAT + PI_CHECK) * lls * (1.0 / N_POS)
                + nls / jnp.maximum(ncs, 1.0)
                + fls / jnp.maximum(N_UNL - ncs, 1.0))
        outb[...] = loss
        pltpu.sync_copy(outb, out)


def kernel(y_pred, y_true, pos_idx, unlabeled_idx, edge_index):
    del y_true
    posp = jnp.zeros((PPAD,), jnp.int32).at[:N_POS].set(pos_idx)
    unlp = jnp.zeros((UPAD,), jnp.int32).at[:N_UNL].set(unlabeled_idx)
    zeros = jnp.zeros((NPAD,), jnp.float32)
    out = _sc_loss(y_pred, posp, unlp, edge_index.reshape(-1), zeros)
    return out[0]


# speedup vs baseline: 291.4185x; 1.8039x over previous
import functools

import jax
import jax.numpy as jnp
from jax import lax
from jax.experimental import pallas as pl
from jax.experimental.pallas import tpu as pltpu
from jax.experimental.pallas import tpu_sc as plsc

N_NODES = 100000
N_EDGES = 3200000
N_POS = 5000
N_UNL = 95000
PI_HAT = 0.6
PI_CHECK = 0.4

L = 16
NC = 2
NT = 16
NW = NC * NT
NPAD = 102400
NODE_PER_TILE = NPAD // NT
E_PER_W = N_EDGES // NW
ECHUNK = 10000
NCHUNK = E_PER_W // ECHUNK
PPAD = 5008
UPAD = 95232
U_PER_TILE = UPAD // NT

_mesh2 = plsc.VectorSubcoreMesh(
    core_axis_name="c", subcore_axis_name="s", num_cores=NC)
_mesh1 = plsc.VectorSubcoreMesh(
    core_axis_name="c", subcore_axis_name="s", num_cores=1)
_params = pltpu.CompilerParams(needs_layout_passes=False)

_EDGE_SCRATCH = [
    pltpu.VMEM((ECHUNK,), jnp.int32),
    pltpu.VMEM((ECHUNK,), jnp.int32),
    pltpu.VMEM((ECHUNK,), jnp.int32),
    pltpu.VMEM((ECHUNK,), jnp.int32),
    pltpu.VMEM((ECHUNK,), jnp.float32),
    pltpu.VMEM((NODE_PER_TILE,), jnp.float32),
    pltpu.VMEM((NODE_PER_TILE,), jnp.float32),
    pltpu.SemaphoreType.DMA,
    pltpu.SemaphoreType.DMA,
    pltpu.SemaphoreType.DMA,
    pltpu.SemaphoreType.DMA,
]


def _edge_round(edges, src_arr, dst_arr, wid,
                srcb0, srcb1, dstb0, dstb1, valb, sems):
    srcbs, dstbs = (srcb0, srcb1), (dstb0, dstb1)
    ebase = wid * E_PER_W

    def issue(i):
        k = i % 2
        off = pl.multiple_of(ebase + i * ECHUNK, 8)
        offd = pl.multiple_of(N_EDGES + ebase + i * ECHUNK, 8)
        ds = pltpu.async_copy(edges.at[pl.ds(off, ECHUNK)], srcbs[k], sems[k])
        dd = pltpu.async_copy(edges.at[pl.ds(offd, ECHUNK)], dstbs[k],
                              sems[2 + k])
        return ds, dd

    pend = issue(0)
    for i in range(NCHUNK):
        nxt = issue(i + 1) if i + 1 < NCHUNK else None
        pend[0].wait()
        pend[1].wait()
        k = i % 2
        pltpu.sync_copy(src_arr.at[srcbs[k]], valb)
        pltpu.sync_copy(valb, dst_arr.at[dstbs[k]], add=True)
        pend = nxt


@functools.partial(
    pl.kernel,
    out_type=[jax.ShapeDtypeStruct((NPAD,), jnp.float32),
              jax.ShapeDtypeStruct((NPAD,), jnp.float32)],
    mesh=_mesh2,
    compiler_params=_params,
    scratch_types=_EDGE_SCRATCH + [
        pltpu.VMEM((PPAD,), jnp.int32),
        pltpu.VMEM((PPAD,), jnp.float32),
        pltpu.VMEM_SHARED((NPAD,), jnp.float32),
        pltpu.VMEM_SHARED((NPAD,), jnp.float32),
    ],
)
def _sc_round1(posi, edges, zeros, outA, outB,
               srcb0, srcb1, dstb0, dstb1, valb, mb1, mb2,
               sem0, sem1, sem2, sem3, pidx, pval, P, S1):
    c = lax.axis_index("c")
    s = lax.axis_index("s")
    wid = s * NC + c
    nslc = pl.ds(s * NODE_PER_TILE, NODE_PER_TILE)

    pltpu.sync_copy(zeros.at[nslc], P.at[nslc])
    pltpu.sync_copy(zeros.at[nslc], S1.at[nslc])

    @pl.when(s == 0)
    def _():
        pltpu.sync_copy(posi, pidx)

        def fill(j, cc):
            off = pl.multiple_of(j * L, L)
            pos = j * L + lax.iota(jnp.int32, L)
            pval[pl.ds(off, L)] = jnp.where(pos < N_POS, 1.0, 0.0)
            return cc

        lax.fori_loop(0, PPAD // L, fill, 0)
        pltpu.sync_copy(pval, P.at[pidx], add=True)

    plsc.subcore_barrier()

    _edge_round(edges, P, S1, wid, srcb0, srcb1, dstb0, dstb1, valb,
                (sem0, sem1, sem2, sem3))
    plsc.subcore_barrier()

    pltpu.sync_copy(S1.at[nslc], mb1)

    @pl.when(c == 0)
    def _():
        pltpu.sync_copy(P.at[nslc], mb2)

        def addb(j, cc):
            off = pl.multiple_of(j * L, L)
            mb1[pl.ds(off, L)] = mb1[pl.ds(off, L)] + mb2[pl.ds(off, L)]
            return cc

        lax.fori_loop(0, NODE_PER_TILE // L, addb, 0)
        pltpu.sync_copy(mb1, outA.at[nslc])

    @pl.when(c == 1)
    def _():
        pltpu.sync_copy(mb1, outB.at[nslc])


@functools.partial(
    pl.kernel,
    out_type=[jax.ShapeDtypeStruct((NPAD,), jnp.float32),
              jax.ShapeDtypeStruct((NPAD,), jnp.float32),
              jax.ShapeDtypeStruct((NPAD,), jnp.float32)],
    mesh=_mesh2,
    compiler_params=_params,
    scratch_types=_EDGE_SCRATCH + [
        pltpu.VMEM_SHARED((NPAD,), jnp.float32),
        pltpu.VMEM_SHARED((NPAD,), jnp.float32),
    ],
)
def _sc_round2(edges, a, b, zeros, outS1G, outS2A, outS2B,
               srcb0, srcb1, dstb0, dstb1, valb, mb1, mb2,
               sem0, sem1, sem2, sem3, S1G, S2):
    c = lax.axis_index("c")
    s = lax.axis_index("s")
    wid = s * NC + c
    nslc = pl.ds(s * NODE_PER_TILE, NODE_PER_TILE)

    pltpu.sync_copy(zeros.at[nslc], S2.at[nslc])

    pltpu.sync_copy(a.at[nslc], mb1)
    pltpu.sync_copy(b.at[nslc], mb2)

    def addb(j, cc):
        off = pl.multiple_of(j * L, L)
        mb1[pl.ds(off, L)] = mb1[pl.ds(off, L)] + mb2[pl.ds(off, L)]
        return cc

    lax.fori_loop(0, NODE_PER_TILE // L, addb, 0)
    pltpu.sync_copy(mb1, S1G.at[nslc])

    @pl.when(c == 0)
    def _():
        pltpu.sync_copy(mb1, outS1G.at[nslc])

    plsc.subcore_barrier()

    _edge_round(edges, S1G, S2, wid, srcb0, srcb1, dstb0, dstb1, valb,
                (sem0, sem1, sem2, sem3))
    plsc.subcore_barrier()

    @pl.when(c == 0)
    def _():
        pltpu.sync_copy(S2.at[nslc], outS2A.at[nslc])

    @pl.when(c == 1)
    def _():
        pltpu.sync_copy(S2.at[nslc], outS2B.at[nslc])


@functools.partial(
    pl.kernel,
    out_type=jax.ShapeDtypeStruct((L,), jnp.float32),
    mesh=_mesh1,
    compiler_params=_params,
    scratch_types=[
        pltpu.VMEM((PPAD,), jnp.int32),
        pltpu.VMEM((PPAD,), jnp.float32),
        pltpu.VMEM((NODE_PER_TILE,), jnp.float32),
        pltpu.VMEM((NODE_PER_TILE,), jnp.float32),
        pltpu.VMEM((U_PER_TILE,), jnp.int32),
        pltpu.VMEM((U_PER_TILE,), jnp.float32),
        pltpu.VMEM((U_PER_TILE,), jnp.float32),
        pltpu.VMEM((4 * L,), jnp.float32),
        pltpu.VMEM((4 * L,), jnp.int32),
        pltpu.VMEM((8 * L,), jnp.float32),
        pltpu.VMEM((L,), jnp.float32),
        pltpu.VMEM_SHARED((NPAD,), jnp.float32),
        pltpu.VMEM_SHARED((8 * L,), jnp.float32),
    ],
)
def _sc_loss(ypred, posi, unli, s1g, s2a, s2b, zeros, out,
             pidx, pval, mb1, mb2, uidx, yub, fb, accb, aidx, redb, outb,
             REACH, SACC):
    w = lax.axis_index("s")
    z = jnp.zeros((L,), jnp.float32)
    nslc = pl.ds(w * NODE_PER_TILE, NODE_PER_TILE)

    pltpu.sync_copy(s1g.at[nslc], mb1)
    pltpu.sync_copy(s2a.at[nslc], mb2)

    def addb(j, cc):
        off = pl.multiple_of(j * L, L)
        mb1[pl.ds(off, L)] = mb1[pl.ds(off, L)] + mb2[pl.ds(off, L)]
        return cc

    lax.fori_loop(0, NODE_PER_TILE // L, addb, 0)
    pltpu.sync_copy(s2b.at[nslc], mb2)
    lax.fori_loop(0, NODE_PER_TILE // L, addb, 0)
    pltpu.sync_copy(mb1, REACH.at[nslc])

    @pl.when(w == 0)
    def _():
        pltpu.sync_copy(zeros.at[pl.ds(0, 8 * L)], SACC)

    plsc.subcore_barrier()

    ub = w * U_PER_TILE
    uslc = pl.ds(ub, U_PER_TILE)
    pltpu.sync_copy(unli.at[uslc], uidx)
    pltpu.sync_copy(ypred.at[uidx], yub)
    pltpu.sync_copy(REACH.at[uidx], fb)

    def lbody(j, carry):
        nc_, nl, fl = carry
        off = pl.multiple_of(j * L, L)
        gpos = ub + j * L + lax.iota(jnp.int32, L)
        valid = gpos < N_UNL
        y = yub[pl.ds(off, L)]
        near = (fb[pl.ds(off, L)] > 0.0) & valid
        far = valid & jnp.logical_not(near)
        nearf = jnp.where(near, 1.0, 0.0)
        farf = jnp.where(far, 1.0, 0.0)
        return (nc_ + nearf,
                nl + nearf * jnp.abs(y - PI_HAT),
                fl + farf * jnp.abs(y - PI_CHECK))

    nc_, nl, fl = lax.fori_loop(0, U_PER_TILE // L, lbody, (z, z, z))
    accb[pl.ds(0, L)] = nc_
    accb[pl.ds(L, L)] = nl
    accb[pl.ds(2 * L, L)] = fl
    accb[pl.ds(3 * L, L)] = z

    @pl.when(w == 0)
    def _():
        pltpu.sync_copy(posi, pidx)
        pltpu.sync_copy(ypred.at[pidx], pval)

        def pbody(j, carry):
            off = pl.multiple_of(j * L, L)
            pos = j * L + lax.iota(jnp.int32, L)
            vf = jnp.where(pos < N_POS, 1.0, 0.0)
            return carry + vf * jnp.abs(pval[pl.ds(off, L)] - 1.0)

        accb[pl.ds(3 * L, L)] = lax.fori_loop(0, PPAD // L, pbody, z)

    def ibody(j, cc):
        off = pl.multiple_of(j * L, L)
        aidx[pl.ds(off, L)] = j * L + lax.iota(jnp.int32, L)
        return cc

    lax.fori_loop(0, 4, ibody, 0)
    pltpu.sync_copy(accb, SACC.at[aidx], add=True)
    plsc.subcore_barrier()

    @pl.when(w == 0)
    def _():
        pltpu.sync_copy(SACC, redb)
        ncs = z + jnp.sum(redb[pl.ds(0, L)])
        nls = z + jnp.sum(redb[pl.ds(L, L)])
        fls = z + jnp.sum(redb[pl.ds(2 * L, L)])
        lls = z + jnp.sum(redb[pl.ds(3 * L, L)])
        loss = (2.0 * (PI_HAT + PI_CHECK) * lls * (1.0 / N_POS)
                + nls / jnp.maximum(ncs, 1.0)
                + fls / jnp.maximum(N_UNL - ncs, 1.0))
        outb[...] = loss
        pltpu.sync_copy(outb, out)


def kernel(y_pred, y_true, pos_idx, unlabeled_idx, edge_index):
    del y_true
    posp = jnp.zeros((PPAD,), jnp.int32).at[:N_POS].set(pos_idx)
    unlp = jnp.zeros((UPAD,), jnp.int32).at[:N_UNL].set(unlabeled_idx)
    zeros = jnp.zeros((NPAD,), jnp.float32)
    edges = edge_index.reshape(-1)
    a, b = _sc_round1(posp, edges, zeros)
    s1g, s2a, s2b = _sc_round2(edges, a, b, zeros)
    out = _sc_loss(y_pred, posp, unlp, s1g, s2a, s2b, zeros)
    return out[0]


# speedup vs baseline: 298.8941x; 1.0257x over previous
import functools

import jax
import jax.numpy as jnp
from jax import lax
from jax.experimental import pallas as pl
from jax.experimental.pallas import tpu as pltpu
from jax.experimental.pallas import tpu_sc as plsc

N_NODES = 100000
N_EDGES = 3200000
N_POS = 5000
N_UNL = 95000
PI_HAT = 0.6
PI_CHECK = 0.4

L = 16
NC = 2
NT = 16
NW = NC * NT
NPAD = 102400
NODE_PER_TILE = NPAD // NT
E_PER_W = N_EDGES // NW
ECHUNK = 10000
NCHUNK = E_PER_W // ECHUNK
PPAD = 5008
UPAD = 95232
U_PER_TILE = UPAD // NT
U_PER_W = UPAD // NW

_mesh2 = plsc.VectorSubcoreMesh(
    core_axis_name="c", subcore_axis_name="s", num_cores=NC)
_mesh1 = plsc.VectorSubcoreMesh(
    core_axis_name="c", subcore_axis_name="s", num_cores=1)
_params = pltpu.CompilerParams(needs_layout_passes=False)

_EDGE_SCRATCH = [
    pltpu.VMEM((ECHUNK,), jnp.int32),
    pltpu.VMEM((ECHUNK,), jnp.int32),
    pltpu.VMEM((ECHUNK,), jnp.int32),
    pltpu.VMEM((ECHUNK,), jnp.int32),
    pltpu.VMEM((ECHUNK,), jnp.float32),
    pltpu.VMEM((NODE_PER_TILE,), jnp.float32),
    pltpu.VMEM((NODE_PER_TILE,), jnp.float32),
    pltpu.SemaphoreType.DMA,
    pltpu.SemaphoreType.DMA,
    pltpu.SemaphoreType.DMA,
    pltpu.SemaphoreType.DMA,
]


def _edge_round(edges, src_arr, dst_arr, wid,
                srcb0, srcb1, dstb0, dstb1, valb, sems):
    srcbs, dstbs = (srcb0, srcb1), (dstb0, dstb1)
    ebase = wid * E_PER_W

    def issue(i):
        k = i % 2
        off = pl.multiple_of(ebase + i * ECHUNK, 8)
        offd = pl.multiple_of(N_EDGES + ebase + i * ECHUNK, 8)
        ds = pltpu.async_copy(edges.at[pl.ds(off, ECHUNK)], srcbs[k], sems[k])
        dd = pltpu.async_copy(edges.at[pl.ds(offd, ECHUNK)], dstbs[k],
                              sems[2 + k])
        return ds, dd

    pend = issue(0)
    for i in range(NCHUNK):
        nxt = issue(i + 1) if i + 1 < NCHUNK else None
        pend[0].wait()
        pend[1].wait()
        k = i % 2
        pltpu.sync_copy(src_arr.at[srcbs[k]], valb)
        pltpu.sync_copy(valb, dst_arr.at[dstbs[k]], add=True)
        pend = nxt


@functools.partial(
    pl.kernel,
    out_type=[jax.ShapeDtypeStruct((NPAD,), jnp.float32),
              jax.ShapeDtypeStruct((NPAD,), jnp.float32)],
    mesh=_mesh2,
    compiler_params=_params,
    scratch_types=_EDGE_SCRATCH + [
        pltpu.VMEM((PPAD,), jnp.int32),
        pltpu.VMEM((PPAD,), jnp.float32),
        pltpu.VMEM_SHARED((NPAD,), jnp.float32),
        pltpu.VMEM_SHARED((NPAD,), jnp.float32),
    ],
)
def _sc_round1(posi, edges, zeros, outA, outB,
               srcb0, srcb1, dstb0, dstb1, valb, mb1, mb2,
               sem0, sem1, sem2, sem3, pidx, pval, P, S1):
    c = lax.axis_index("c")
    s = lax.axis_index("s")
    wid = s * NC + c
    nslc = pl.ds(s * NODE_PER_TILE, NODE_PER_TILE)

    pltpu.sync_copy(zeros.at[nslc], P.at[nslc])
    pltpu.sync_copy(zeros.at[nslc], S1.at[nslc])

    @pl.when(s == 0)
    def _():
        pltpu.sync_copy(posi, pidx)

        def fill(j, cc):
            off = pl.multiple_of(j * L, L)
            pos = j * L + lax.iota(jnp.int32, L)
            pval[pl.ds(off, L)] = jnp.where(pos < N_POS, 1.0, 0.0)
            return cc

        lax.fori_loop(0, PPAD // L, fill, 0)
        pltpu.sync_copy(pval, P.at[pidx], add=True)

    plsc.subcore_barrier()

    _edge_round(edges, P, S1, wid, srcb0, srcb1, dstb0, dstb1, valb,
                (sem0, sem1, sem2, sem3))
    plsc.subcore_barrier()

    pltpu.sync_copy(S1.at[nslc], mb1)

    @pl.when(c == 0)
    def _():
        pltpu.sync_copy(P.at[nslc], mb2)

        def addb(j, cc):
            off = pl.multiple_of(j * L, L)
            mb1[pl.ds(off, L)] = mb1[pl.ds(off, L)] + mb2[pl.ds(off, L)]
            return cc

        lax.fori_loop(0, NODE_PER_TILE // L, addb, 0)
        pltpu.sync_copy(mb1, outA.at[nslc])

    @pl.when(c == 1)
    def _():
        pltpu.sync_copy(mb1, outB.at[nslc])


@functools.partial(
    pl.kernel,
    out_type=[jax.ShapeDtypeStruct((NPAD,), jnp.float32),
              jax.ShapeDtypeStruct((NPAD,), jnp.float32),
              jax.ShapeDtypeStruct((NPAD,), jnp.float32),
              jax.ShapeDtypeStruct((UPAD,), jnp.float32)],
    mesh=_mesh2,
    compiler_params=_params,
    scratch_types=_EDGE_SCRATCH + [
        pltpu.VMEM((U_PER_W,), jnp.int32),
        pltpu.VMEM((U_PER_W,), jnp.float32),
        pltpu.SemaphoreType.DMA,
        pltpu.VMEM_SHARED((NPAD,), jnp.float32),
        pltpu.VMEM_SHARED((NPAD,), jnp.float32),
    ],
)
def _sc_round2(edges, a, b, zeros, ypred, unli, outS1G, outS2A, outS2B, outY,
               srcb0, srcb1, dstb0, dstb1, valb, mb1, mb2,
               sem0, sem1, sem2, sem3, uidx2, yub2, semy, S1G, S2):
    c = lax.axis_index("c")
    s = lax.axis_index("s")
    wid = s * NC + c
    nslc = pl.ds(s * NODE_PER_TILE, NODE_PER_TILE)

    ub32 = wid * U_PER_W
    pltpu.sync_copy(unli.at[pl.ds(ub32, U_PER_W)], uidx2)
    ydesc = pltpu.async_copy(ypred.at[uidx2], yub2, semy)

    pltpu.sync_copy(zeros.at[nslc], S2.at[nslc])

    pltpu.sync_copy(a.at[nslc], mb1)
    pltpu.sync_copy(b.at[nslc], mb2)

    def addb(j, cc):
        off = pl.multiple_of(j * L, L)
        mb1[pl.ds(off, L)] = mb1[pl.ds(off, L)] + mb2[pl.ds(off, L)]
        return cc

    lax.fori_loop(0, NODE_PER_TILE // L, addb, 0)
    pltpu.sync_copy(mb1, S1G.at[nslc])

    @pl.when(c == 0)
    def _():
        pltpu.sync_copy(mb1, outS1G.at[nslc])

    plsc.subcore_barrier()

    _edge_round(edges, S1G, S2, wid, srcb0, srcb1, dstb0, dstb1, valb,
                (sem0, sem1, sem2, sem3))
    plsc.subcore_barrier()

    @pl.when(c == 0)
    def _():
        pltpu.sync_copy(S2.at[nslc], outS2A.at[nslc])

    @pl.when(c == 1)
    def _():
        pltpu.sync_copy(S2.at[nslc], outS2B.at[nslc])

    ydesc.wait()
    pltpu.sync_copy(yub2, outY.at[pl.ds(ub32, U_PER_W)])


@functools.partial(
    pl.kernel,
    out_type=jax.ShapeDtypeStruct((L,), jnp.float32),
    mesh=_mesh1,
    compiler_params=_params,
    scratch_types=[
        pltpu.VMEM((PPAD,), jnp.int32),
        pltpu.VMEM((PPAD,), jnp.float32),
        pltpu.VMEM((NODE_PER_TILE,), jnp.float32),
        pltpu.VMEM((NODE_PER_TILE,), jnp.float32),
        pltpu.VMEM((U_PER_TILE,), jnp.int32),
        pltpu.VMEM((U_PER_TILE,), jnp.float32),
        pltpu.VMEM((U_PER_TILE,), jnp.float32),
        pltpu.VMEM((4 * L,), jnp.float32),
        pltpu.VMEM((4 * L,), jnp.int32),
        pltpu.VMEM((8 * L,), jnp.float32),
        pltpu.VMEM((L,), jnp.float32),
        pltpu.VMEM_SHARED((NPAD,), jnp.float32),
        pltpu.VMEM_SHARED((8 * L,), jnp.float32),
    ],
)
def _sc_loss(ypred, posi, unli, s1g, s2a, s2b, yflat, zeros, out,
             pidx, pval, mb1, mb2, uidx, yub, fb, accb, aidx, redb, outb,
             REACH, SACC):
    w = lax.axis_index("s")
    z = jnp.zeros((L,), jnp.float32)
    nslc = pl.ds(w * NODE_PER_TILE, NODE_PER_TILE)

    pltpu.sync_copy(s1g.at[nslc], mb1)
    pltpu.sync_copy(s2a.at[nslc], mb2)

    def addb(j, cc):
        off = pl.multiple_of(j * L, L)
        mb1[pl.ds(off, L)] = mb1[pl.ds(off, L)] + mb2[pl.ds(off, L)]
        return cc

    lax.fori_loop(0, NODE_PER_TILE // L, addb, 0)
    pltpu.sync_copy(s2b.at[nslc], mb2)
    lax.fori_loop(0, NODE_PER_TILE // L, addb, 0)
    pltpu.sync_copy(mb1, REACH.at[nslc])

    @pl.when(w == 0)
    def _():
        pltpu.sync_copy(zeros.at[pl.ds(0, 8 * L)], SACC)

    plsc.subcore_barrier()

    ub = w * U_PER_TILE
    uslc = pl.ds(ub, U_PER_TILE)
    pltpu.sync_copy(unli.at[uslc], uidx)
    pltpu.sync_copy(yflat.at[uslc], yub)
    pltpu.sync_copy(REACH.at[uidx], fb)

    def lbody(j, carry):
        nc_, nl, fl = carry
        off = pl.multiple_of(j * L, L)
        gpos = ub + j * L + lax.iota(jnp.int32, L)
        valid = gpos < N_UNL
        y = yub[pl.ds(off, L)]
        near = (fb[pl.ds(off, L)] > 0.0) & valid
        far = valid & jnp.logical_not(near)
        nearf = jnp.where(near, 1.0, 0.0)
        farf = jnp.where(far, 1.0, 0.0)
        return (nc_ + nearf,
                nl + nearf * jnp.abs(y - PI_HAT),
                fl + farf * jnp.abs(y - PI_CHECK))

    nc_, nl, fl = lax.fori_loop(0, U_PER_TILE // L, lbody, (z, z, z))
    accb[pl.ds(0, L)] = nc_
    accb[pl.ds(L, L)] = nl
    accb[pl.ds(2 * L, L)] = fl
    accb[pl.ds(3 * L, L)] = z

    @pl.when(w == 0)
    def _():
        pltpu.sync_copy(posi, pidx)
        pltpu.sync_copy(ypred.at[pidx], pval)

        def pbody(j, carry):
            off = pl.multiple_of(j * L, L)
            pos = j * L + lax.iota(jnp.int32, L)
            vf = jnp.where(pos < N_POS, 1.0, 0.0)
            return carry + vf * jnp.abs(pval[pl.ds(off, L)] - 1.0)

        accb[pl.ds(3 * L, L)] = lax.fori_loop(0, PPAD // L, pbody, z)

    def ibody(j, cc):
        off = pl.multiple_of(j * L, L)
        aidx[pl.ds(off, L)] = j * L + lax.iota(jnp.int32, L)
        return cc

    lax.fori_loop(0, 4, ibody, 0)
    pltpu.sync_copy(accb, SACC.at[aidx], add=True)
    plsc.subcore_barrier()

    @pl.when(w == 0)
    def _():
        pltpu.sync_copy(SACC, redb)
        ncs = z + jnp.sum(redb[pl.ds(0, L)])
        nls = z + jnp.sum(redb[pl.ds(L, L)])
        fls = z + jnp.sum(redb[pl.ds(2 * L, L)])
        lls = z + jnp.sum(redb[pl.ds(3 * L, L)])
        loss = (2.0 * (PI_HAT + PI_CHECK) * lls * (1.0 / N_POS)
                + nls / jnp.maximum(ncs, 1.0)
                + fls / jnp.maximum(N_UNL - ncs, 1.0))
        outb[...] = loss
        pltpu.sync_copy(outb, out)


def kernel(y_pred, y_true, pos_idx, unlabeled_idx, edge_index):
    del y_true
    posp = jnp.zeros((PPAD,), jnp.int32).at[:N_POS].set(pos_idx)
    unlp = jnp.zeros((UPAD,), jnp.int32).at[:N_UNL].set(unlabeled_idx)
    zeros = jnp.zeros((NPAD,), jnp.float32)
    edges = edge_index.reshape(-1)
    a, b = _sc_round1(posp, edges, zeros)
    s1g, s2a, s2b, yflat = _sc_round2(edges, a, b, zeros, y_pred, unlp)
    out = _sc_loss(y_pred, posp, unlp, s1g, s2a, s2b, yflat, zeros)
    return out[0]
